# Initial kernel scaffold; baseline (speedup 1.0000x reference)
import jax
import jax.numpy as jnp
from jax.experimental import pallas as pl


def kernel(feat, edge_index_r0, edge_index_r1, edge_index_r2, edge_index_r3, edge_weight_r0, edge_weight_r1, edge_weight_r2, edge_weight_r3, node_fc_W, node_fc_b, rela_fc_W, rela_fc_b, W1, b1, W2, b2):
    raise NotImplementedError("write your pallas kernel here")


# speedup vs baseline: 2.1153x; 2.1153x over previous
import functools

import jax
import jax.numpy as jnp
from jax import lax
from jax.experimental import pallas as pl
from jax.experimental.pallas import tpu as pltpu
from jax.experimental.pallas import tpu_sc as plsc

N = 10000
D = 128
R = 4
E = 160000
NP = 10240
NS = 16
NC = 2
TPB = E // NS
BB = 80
NBLK = TPB // BB
ROWS_PT = NP // NS


def _mesh():
    return plsc.VectorSubcoreMesh(core_axis_name="c", subcore_axis_name="s")


@functools.partial(
    pl.kernel,
    out_type=jax.ShapeDtypeStruct((R * 2 * NP,), jnp.float32),
    mesh=_mesh(),
    scratch_types=[
        pltpu.VMEM_SHARED((4 * NP,), jnp.float32),
        pltpu.VMEM((BB,), jnp.int32),
        pltpu.VMEM((BB,), jnp.float32),
        pltpu.VMEM((ROWS_PT,), jnp.float32),
    ],
)
def _deg_kernel(idx_hbm, out_hbm, acc, idxv, onesv, zv):
    c = lax.axis_index("c")
    s = lax.axis_index("s")

    def zi(i, _):
        zv[pl.ds(i * 16, 16)] = jnp.zeros((16,), jnp.float32)
        return 0

    lax.fori_loop(0, ROWS_PT // 16, zi, 0)
    for j in range(BB // 16):
        onesv[pl.ds(j * 16, 16)] = jnp.ones((16,), jnp.float32)
    for z in range(4):
        pltpu.sync_copy(zv, acc.at[pl.ds((s * 4 + z) * ROWS_PT, ROWS_PT)])
    plsc.subcore_barrier()
    for rel_i in range(2):
        r = c + 2 * rel_i
        for side in range(2):
            segbase = (r * 2 + side) * E + s * TPB

            def blk(b, _):
                pltpu.sync_copy(idx_hbm.at[pl.ds(segbase + b * BB, BB)], idxv)
                pltpu.sync_copy(onesv, acc.at[idxv], add=True)
                return 0

            lax.fori_loop(0, NBLK, blk, 0)
    plsc.subcore_barrier()
    for rel_i in range(2):
        r = c + 2 * rel_i
        for side in range(2):
            k = rel_i * 2 + side
            pltpu.sync_copy(
                acc.at[pl.ds(k * NP + s * ROWS_PT, ROWS_PT)],
                out_hbm.at[pl.ds((r * 2 + side) * NP + s * ROWS_PT, ROWS_PT)],
            )


@functools.partial(
    pl.kernel,
    out_type=jax.ShapeDtypeStruct((R * NP, D), jnp.float32),
    mesh=_mesh(),
    scratch_types=[
        pltpu.VMEM_SHARED((NP, D), jnp.float32),
        pltpu.VMEM((BB,), jnp.int32),
        pltpu.VMEM((BB,), jnp.int32),
        pltpu.VMEM((BB, D), jnp.float32),
        pltpu.VMEM((BB, D), jnp.float32),
        pltpu.VMEM((128, D), jnp.float32),
        pltpu.SemaphoreType.DMA,
    ],
)
def _edge_kernel(hs_hbm, rw_hbm, sidx_hbm, didx_hbm, agg_hbm,
                 acc, siv, div, hrow, rrow, zbuf, gsem):
    c = lax.axis_index("c")
    s = lax.axis_index("s")

    def zi(i, _):
        for j in range(D // 16):
            zbuf[i, pl.ds(j * 16, 16)] = jnp.zeros((16,), jnp.float32)
        return 0

    lax.fori_loop(0, 128, zi, 0)
    for rel_i in range(2):
        r = c + 2 * rel_i
        for z in range(ROWS_PT // 128):
            pltpu.sync_copy(zbuf, acc.at[pl.ds(s * ROWS_PT + z * 128, 128)])
        plsc.subcore_barrier()

        def blk(b, _):
            eoff = r * E + s * TPB + b * BB
            pltpu.sync_copy(sidx_hbm.at[pl.ds(eoff, BB)], siv)
            pltpu.sync_copy(didx_hbm.at[pl.ds(eoff, BB)], div)
            pltpu.async_copy(hs_hbm.at[siv], hrow, gsem).wait()
            pltpu.sync_copy(rw_hbm.at[pl.ds(eoff, BB)], rrow)

            def mrow(i, _):
                for j in range(D // 16):
                    hrow[i, pl.ds(j * 16, 16)] = (
                        hrow[i, pl.ds(j * 16, 16)] * rrow[i, pl.ds(j * 16, 16)]
                    )
                return 0

            lax.fori_loop(0, BB, mrow, 0)
            pltpu.sync_copy(hrow, acc.at[div], add=True)
            return 0

        lax.fori_loop(0, NBLK, blk, 0)
        plsc.subcore_barrier()
        pltpu.sync_copy(
            acc.at[pl.ds(s * ROWS_PT, ROWS_PT)],
            agg_hbm.at[pl.ds(r * NP + s * ROWS_PT, ROWS_PT)],
        )
        if rel_i == 0:
            plsc.subcore_barrier()


def _matmul_bias_body(x_ref, w_ref, b_ref, o_ref):
    o_ref[...] = (
        jnp.dot(x_ref[...], w_ref[...], preferred_element_type=jnp.float32)
        + b_ref[...]
    )


def _rw_call(ews_flat, wT, b):
    BA = 2560
    return pl.pallas_call(
        _matmul_bias_body,
        grid=(R * E // BA,),
        in_specs=[
            pl.BlockSpec((BA, D), lambda i: (i, 0)),
            pl.BlockSpec((D, D), lambda i: (0, 0)),
            pl.BlockSpec((1, D), lambda i: (0, 0)),
        ],
        out_specs=pl.BlockSpec((BA, D), lambda i: (i, 0)),
        out_shape=jax.ShapeDtypeStruct((R * E, D), jnp.float32),
    )(ews_flat, wT, b)


def _h0_call(feat, wT, b):
    return pl.pallas_call(
        _matmul_bias_body,
        out_shape=jax.ShapeDtypeStruct((N, D), jnp.float32),
    )(feat, wT, b)


def _prep_body(h0_ref, dt_ref, sc_ref, hs_ref):
    sc = lax.rsqrt(jnp.maximum(dt_ref[...], 1.0))
    sc_ref[...] = sc
    h = h0_ref[...]
    for r in range(R):
        hs_ref[r, 0:N, :] = h * sc[0:N, 2 * r:2 * r + 1]


def _prep_call(h0, degs_t):
    return pl.pallas_call(
        _prep_body,
        out_shape=[
            jax.ShapeDtypeStruct((NP, 8), jnp.float32),
            jax.ShapeDtypeStruct((R, NP, D), jnp.float32),
        ],
    )(h0, degs_t)


_BN = 1000


def _combine_mid_body(agg_ref, w_ref, b_ref, sc_ref, hs2_ref):
    sc = sc_ref[...]
    acc = jnp.zeros((_BN, D), jnp.float32)
    for r in range(R):
        acc = acc + (
            jnp.dot(agg_ref[r], w_ref[r], preferred_element_type=jnp.float32)
            * sc[:, 2 * r + 1:2 * r + 2]
        )
    emb = acc * (1.0 / R) + jnp.sum(b_ref[...], axis=0, keepdims=True) * (1.0 / R)
    for r in range(R):
        hs2_ref[r] = emb * sc[:, 2 * r:2 * r + 1]


def _combine_mid_call(agg3, W, b, scales):
    return pl.pallas_call(
        _combine_mid_body,
        grid=(N // _BN,),
        in_specs=[
            pl.BlockSpec((R, _BN, D), lambda i: (0, i, 0)),
            pl.BlockSpec((R, D, D), lambda i: (0, 0, 0)),
            pl.BlockSpec((R, D), lambda i: (0, 0)),
            pl.BlockSpec((_BN, 8), lambda i: (i, 0)),
        ],
        out_specs=pl.BlockSpec((R, _BN, D), lambda i: (0, i, 0)),
        out_shape=jax.ShapeDtypeStruct((R, NP, D), jnp.float32),
    )(agg3, W, b, scales)


def _combine_final_body(agg_ref, w_ref, b_ref, sc_ref, out_ref):
    sc = sc_ref[...]
    acc = jnp.zeros((_BN, D), jnp.float32)
    for r in range(R):
        acc = acc + (
            jnp.dot(agg_ref[r], w_ref[r], preferred_element_type=jnp.float32)
            * sc[:, 2 * r + 1:2 * r + 2]
        )
    out_ref[...] = acc * (1.0 / R) + jnp.sum(b_ref[...], axis=0, keepdims=True) * (1.0 / R)


def _combine_final_call(agg3, W, b, scales):
    return pl.pallas_call(
        _combine_final_body,
        grid=(N // _BN,),
        in_specs=[
            pl.BlockSpec((R, _BN, D), lambda i: (0, i, 0)),
            pl.BlockSpec((R, D, D), lambda i: (0, 0, 0)),
            pl.BlockSpec((R, D), lambda i: (0, 0)),
            pl.BlockSpec((_BN, 8), lambda i: (i, 0)),
        ],
        out_specs=pl.BlockSpec((_BN, D), lambda i: (i, 0)),
        out_shape=jax.ShapeDtypeStruct((N, D), jnp.float32),
    )(agg3, W, b, scales)


def kernel(feat, edge_index_r0, edge_index_r1, edge_index_r2, edge_index_r3,
           edge_weight_r0, edge_weight_r1, edge_weight_r2, edge_weight_r3,
           node_fc_W, node_fc_b, rela_fc_W, rela_fc_b, W1, b1, W2, b2):
    eis = [edge_index_r0, edge_index_r1, edge_index_r2, edge_index_r3]
    ews = [edge_weight_r0, edge_weight_r1, edge_weight_r2, edge_weight_r3]

    srcs = [eis[r][0].astype(jnp.int32) for r in range(R)]
    dsts = [eis[r][1].astype(jnp.int32) for r in range(R)]
    src_adj = jnp.concatenate([srcs[r] + r * NP for r in range(R)])
    dst_flat = jnp.concatenate(dsts)
    deg_idx = jnp.concatenate(
        [jnp.concatenate([srcs[r] + ((r // 2) * 2) * NP,
                          dsts[r] + ((r // 2) * 2 + 1) * NP])
         for r in range(R)])
    ews_flat = jnp.concatenate(ews, axis=0)

    rw_flat = _rw_call(ews_flat, rela_fc_W.T, rela_fc_b[None, :])
    h0 = _h0_call(feat, node_fc_W.T, node_fc_b[None, :])
    degs_flat = _deg_kernel(deg_idx)
    degs_t = degs_flat.reshape(R * 2, NP).T
    scales, hs1 = _prep_call(h0, degs_t)

    agg1 = _edge_kernel(hs1.reshape(R * NP, D), rw_flat, src_adj, dst_flat)
    hs2 = _combine_mid_call(agg1.reshape(R, NP, D), W1, b1, scales)
    agg2 = _edge_kernel(hs2.reshape(R * NP, D), rw_flat, src_adj, dst_flat)
    emb2 = _combine_final_call(agg2.reshape(R, NP, D), W2, b2, scales)
    return emb2


# speedup vs baseline: 3.9657x; 1.8748x over previous
import functools

import jax
import jax.numpy as jnp
from jax import lax
from jax.experimental import pallas as pl
from jax.experimental.pallas import tpu as pltpu
from jax.experimental.pallas import tpu_sc as plsc

N = 10000
D = 128
R = 4
E = 160000
NP = 10240
NS = 16
NC = 2
TPB = E // NS
BB = 80
NBLK = TPB // BB
ROWS_PT = NP // NS


def _mesh():
    return plsc.VectorSubcoreMesh(core_axis_name="c", subcore_axis_name="s")


@functools.partial(
    pl.kernel,
    out_type=jax.ShapeDtypeStruct((R * 2 * NP,), jnp.float32),
    mesh=_mesh(),
    scratch_types=[
        pltpu.VMEM_SHARED((4 * NP,), jnp.float32),
        pltpu.VMEM((BB,), jnp.int32),
        pltpu.VMEM((BB,), jnp.float32),
        pltpu.VMEM((ROWS_PT,), jnp.float32),
        pltpu.SemaphoreType.DMA,
    ],
)
def _deg_kernel(idx_hbm, out_hbm, acc, idxv, onesv, zv, dsem):
    c = lax.axis_index("c")
    s = lax.axis_index("s")

    def zi(i, _):
        zv[pl.ds(i * 16, 16)] = jnp.zeros((16,), jnp.float32)
        return 0

    lax.fori_loop(0, ROWS_PT // 16, zi, 0)
    for j in range(BB // 16):
        onesv[pl.ds(j * 16, 16)] = jnp.ones((16,), jnp.float32)
    for z in range(4):
        pltpu.sync_copy(zv, acc.at[pl.ds((s * 4 + z) * ROWS_PT, ROWS_PT)])
    plsc.subcore_barrier()
    for rel_i in range(2):
        r = c + 2 * rel_i
        for side in range(2):
            segbase = (r * 2 + side) * E + s * TPB

            def blk(b, _):
                pltpu.sync_copy(idx_hbm.at[pl.ds(segbase + b * BB, BB)], idxv)
                pltpu.async_copy(onesv, acc.at[idxv], dsem, add=True).wait()
                return 0

            lax.fori_loop(0, NBLK, blk, 0)
    plsc.subcore_barrier()
    for k in range(4):
        r = c + 2 * (k // 2)
        side = k % 2
        pltpu.sync_copy(
            acc.at[pl.ds(k * NP + s * ROWS_PT, ROWS_PT)],
            out_hbm.at[pl.ds((r * 2 + side) * NP + s * ROWS_PT, ROWS_PT)],
        )


@functools.partial(
    pl.kernel,
    out_type=jax.ShapeDtypeStruct((R * NP, D), jnp.float32),
    mesh=_mesh(),
    scratch_types=[
        pltpu.VMEM_SHARED((NP, D), jnp.float32),
        pltpu.VMEM((BB,), jnp.int32),
        pltpu.VMEM((BB,), jnp.int32),
        pltpu.VMEM((BB,), jnp.int32),
        pltpu.VMEM((BB,), jnp.int32),
        pltpu.VMEM((BB, D), jnp.float32),
        pltpu.VMEM((BB, D), jnp.float32),
        pltpu.VMEM((BB, D), jnp.float32),
        pltpu.VMEM((BB, D), jnp.float32),
        pltpu.VMEM((16, D), jnp.float32),
        pltpu.SemaphoreType.DMA,
        pltpu.SemaphoreType.DMA,
        pltpu.SemaphoreType.DMA,
        pltpu.SemaphoreType.DMA,
        pltpu.SemaphoreType.DMA,
        pltpu.SemaphoreType.DMA,
        pltpu.SemaphoreType.DMA,
        pltpu.SemaphoreType.DMA,
        pltpu.SemaphoreType.DMA,
        pltpu.SemaphoreType.DMA,
        pltpu.SemaphoreType.DMA,
    ],
)
def _edge_kernel(hs_hbm, rw_hbm, sidx_hbm, didx_hbm, agg_hbm,
                 acc, si0, si1, di0, di1, hrow0, hrow1, rrow0, rrow1, zbuf,
                 i0, i1, j0, j1, g0, g1, r0, r1, s0, s1, zsem):
    c = lax.axis_index("c")
    s = lax.axis_index("s")
    sidxs = (si0, si1)
    didxs = (di0, di1)
    hrows = (hrow0, hrow1)
    rrows = (rrow0, rrow1)
    isems = (i0, i1)
    jsems = (j0, j1)
    gsems = (g0, g1)
    rsems = (r0, r1)
    ssems = (s0, s1)

    def zi(i, _):
        for j in range(D // 16):
            zbuf[i, pl.ds(j * 16, 16)] = jnp.zeros((16,), jnp.float32)
        return 0

    lax.fori_loop(0, 16, zi, 0)

    def wait_idx(dst, sem):
        pltpu.make_async_copy(sidx_hbm.at[pl.ds(0, BB)], dst, sem).wait()

    def wait_rows(dst, sem):
        pltpu.make_async_copy(hs_hbm.at[pl.ds(0, BB)], dst, sem).wait()

    for rel_i in range(2):
        r = c + 2 * rel_i
        eoff = r * E + s * TPB

        def load_sidx(b, slot):
            pltpu.async_copy(sidx_hbm.at[pl.ds(eoff + b * BB, BB)],
                             sidxs[slot], isems[slot])

        def load_didx(b, slot):
            pltpu.async_copy(didx_hbm.at[pl.ds(eoff + b * BB, BB)],
                             didxs[slot], jsems[slot])

        def load_rw(b, slot):
            pltpu.async_copy(rw_hbm.at[pl.ds(eoff + b * BB, BB)],
                             rrows[slot], rsems[slot])

        def zz(z, _):
            pltpu.async_copy(zbuf, acc.at[pl.ds(s * ROWS_PT + z * 16, 16)],
                             zsem)
            return 0

        load_sidx(0, 0)
        load_didx(0, 0)
        load_rw(0, 0)
        lax.fori_loop(0, ROWS_PT // 16, zz, 0)

        def zw(z, _):
            pltpu.make_async_copy(hs_hbm.at[pl.ds(0, 16)], zbuf, zsem).wait()
            return 0

        lax.fori_loop(0, ROWS_PT // 16, zw, 0)
        plsc.subcore_barrier()
        wait_idx(sidxs[0], isems[0])
        pltpu.async_copy(hs_hbm.at[sidxs[0]], hrows[0], gsems[0])
        load_sidx(1, 1)

        def step(b, slot, other, first, has_next, has_next2):
            if not first:
                wait_rows(hrows[other], ssems[other])
            wait_rows(hrows[slot], gsems[slot])
            if has_next:
                load_didx(b + 1, other)
                load_rw(b + 1, other)
                wait_idx(sidxs[other], isems[other])
                pltpu.async_copy(hs_hbm.at[sidxs[other]], hrows[other],
                                 gsems[other])
            if has_next2:
                load_sidx(b + 2, slot)
            wait_rows(rrows[slot], rsems[slot])

            def mrow(i, _):
                for j in range(D // 16):
                    hrows[slot][i, pl.ds(j * 16, 16)] = (
                        hrows[slot][i, pl.ds(j * 16, 16)]
                        * rrows[slot][i, pl.ds(j * 16, 16)]
                    )
                return 0

            lax.fori_loop(0, BB, mrow, 0)
            wait_idx(didxs[slot], jsems[slot])
            pltpu.async_copy(hrows[slot], acc.at[didxs[slot]],
                             ssems[slot], add=True)

        step(0, 0, 1, True, True, True)

        def pair(p, _):
            step(2 * p + 1, 1, 0, False, True, True)
            step(2 * p + 2, 0, 1, False, True, True)
            return 0

        lax.fori_loop(0, (NBLK - 3) // 2, pair, 0)
        step(NBLK - 2, 1, 0, False, True, False)
        step(NBLK - 1, 0, 1, False, False, False)
        wait_rows(hrows[0], ssems[0])
        plsc.subcore_barrier()
        pltpu.sync_copy(
            acc.at[pl.ds(s * ROWS_PT, ROWS_PT)],
            agg_hbm.at[pl.ds(r * NP + s * ROWS_PT, ROWS_PT)],
        )


def _matmul_bias_body(x_ref, w_ref, b_ref, o_ref):
    o_ref[...] = (
        jnp.dot(x_ref[...], w_ref[...], preferred_element_type=jnp.float32)
        + b_ref[...]
    )


def _rw_call(ews_flat, wT, b):
    BA = 2560
    return pl.pallas_call(
        _matmul_bias_body,
        grid=(R * E // BA,),
        in_specs=[
            pl.BlockSpec((BA, D), lambda i: (i, 0)),
            pl.BlockSpec((D, D), lambda i: (0, 0)),
            pl.BlockSpec((1, D), lambda i: (0, 0)),
        ],
        out_specs=pl.BlockSpec((BA, D), lambda i: (i, 0)),
        out_shape=jax.ShapeDtypeStruct((R * E, D), jnp.float32),
    )(ews_flat, wT, b)


def _h0_call(feat, wT, b):
    return pl.pallas_call(
        _matmul_bias_body,
        out_shape=jax.ShapeDtypeStruct((N, D), jnp.float32),
    )(feat, wT, b)


def _prep_body(h0_ref, dt_ref, sc_ref, hs_ref):
    sc = lax.rsqrt(jnp.maximum(dt_ref[...], 1.0))
    sc_ref[...] = sc
    h = h0_ref[...]
    for r in range(R):
        hs_ref[r, 0:N, :] = h * sc[0:N, 2 * r:2 * r + 1]


def _prep_call(h0, degs_t):
    return pl.pallas_call(
        _prep_body,
        out_shape=[
            jax.ShapeDtypeStruct((NP, 8), jnp.float32),
            jax.ShapeDtypeStruct((R, NP, D), jnp.float32),
        ],
    )(h0, degs_t)


_BN = 1000


def _combine_emb(agg_ref, w_ref, b_ref, sc_ref):
    sc = sc_ref[...]
    acc = jnp.zeros((_BN, D), jnp.float32)
    for r in range(R):
        acc = acc + (
            jnp.dot(agg_ref[r], w_ref[r], preferred_element_type=jnp.float32)
            * sc[:, 2 * r + 1:2 * r + 2]
        )
    return acc * (1.0 / R) + jnp.sum(b_ref[...], axis=0, keepdims=True) * (1.0 / R)


def _combine_mid_body(agg_ref, w_ref, b_ref, sc_ref, hs2_ref):
    emb = _combine_emb(agg_ref, w_ref, b_ref, sc_ref)
    sc = sc_ref[...]
    for r in range(R):
        hs2_ref[r] = emb * sc[:, 2 * r:2 * r + 1]


_COMBINE_IN_SPECS = [
    pl.BlockSpec((R, _BN, D), lambda i: (0, i, 0)),
    pl.BlockSpec((R, D, D), lambda i: (0, 0, 0)),
    pl.BlockSpec((R, D), lambda i: (0, 0)),
    pl.BlockSpec((_BN, 8), lambda i: (i, 0)),
]


def _combine_mid_call(agg3, W, b, scales):
    return pl.pallas_call(
        _combine_mid_body,
        grid=(N // _BN,),
        in_specs=_COMBINE_IN_SPECS,
        out_specs=pl.BlockSpec((R, _BN, D), lambda i: (0, i, 0)),
        out_shape=jax.ShapeDtypeStruct((R, NP, D), jnp.float32),
    )(agg3, W, b, scales)


def _combine_final_body(agg_ref, w_ref, b_ref, sc_ref, out_ref):
    out_ref[...] = _combine_emb(agg_ref, w_ref, b_ref, sc_ref)


def _combine_final_call(agg3, W, b, scales):
    return pl.pallas_call(
        _combine_final_body,
        grid=(N // _BN,),
        in_specs=_COMBINE_IN_SPECS,
        out_specs=pl.BlockSpec((_BN, D), lambda i: (i, 0)),
        out_shape=jax.ShapeDtypeStruct((N, D), jnp.float32),
    )(agg3, W, b, scales)


def kernel(feat, edge_index_r0, edge_index_r1, edge_index_r2, edge_index_r3,
           edge_weight_r0, edge_weight_r1, edge_weight_r2, edge_weight_r3,
           node_fc_W, node_fc_b, rela_fc_W, rela_fc_b, W1, b1, W2, b2):
    eis = [edge_index_r0, edge_index_r1, edge_index_r2, edge_index_r3]
    ews = [edge_weight_r0, edge_weight_r1, edge_weight_r2, edge_weight_r3]

    srcs = [eis[r][0].astype(jnp.int32) for r in range(R)]
    dsts = [eis[r][1].astype(jnp.int32) for r in range(R)]
    src_adj = jnp.concatenate([srcs[r] + r * NP for r in range(R)])
    dst_flat = jnp.concatenate(dsts)
    deg_idx = jnp.concatenate(
        [jnp.concatenate([srcs[r] + ((r // 2) * 2) * NP,
                          dsts[r] + ((r // 2) * 2 + 1) * NP])
         for r in range(R)])
    ews_flat = jnp.concatenate(ews, axis=0)

    rw_flat = _rw_call(ews_flat, rela_fc_W.T, rela_fc_b[None, :])
    h0 = _h0_call(feat, node_fc_W.T, node_fc_b[None, :])
    degs_flat = _deg_kernel(deg_idx)
    degs_t = degs_flat.reshape(R * 2, NP).T
    scales, hs1 = _prep_call(h0, degs_t)

    agg1 = _edge_kernel(hs1.reshape(R * NP, D), rw_flat, src_adj, dst_flat)
    hs2 = _combine_mid_call(agg1.reshape(R, NP, D), W1, b1, scales)
    agg2 = _edge_kernel(hs2.reshape(R * NP, D), rw_flat, src_adj, dst_flat)
    return _combine_final_call(agg2.reshape(R, NP, D), W2, b2, scales)


# speedup vs baseline: 3.9692x; 1.0009x over previous
import functools

import jax
import jax.numpy as jnp
from jax import lax
from jax.experimental import pallas as pl
from jax.experimental.pallas import tpu as pltpu
from jax.experimental.pallas import tpu_sc as plsc

N = 10000
D = 128
R = 4
E = 160000
NP = 10240
NS = 16
NC = 2
TPB = E // NS
BB = 80
NBLK = TPB // BB
ROWS_PT = NP // NS


def _mesh():
    return plsc.VectorSubcoreMesh(core_axis_name="c", subcore_axis_name="s")


@functools.partial(
    pl.kernel,
    out_type=jax.ShapeDtypeStruct((R * 2 * NP,), jnp.float32),
    mesh=_mesh(),
    scratch_types=[
        pltpu.VMEM((BB,), jnp.int32),
        pltpu.VMEM((BB,), jnp.int32),
        pltpu.VMEM((BB,), jnp.int32),
        pltpu.VMEM((BB,), jnp.int32),
        pltpu.VMEM_SHARED((4 * NP,), jnp.float32),
        pltpu.VMEM((BB,), jnp.float32),
        pltpu.VMEM((ROWS_PT,), jnp.float32),
        pltpu.SemaphoreType.DMA,
        pltpu.SemaphoreType.DMA,
        pltpu.SemaphoreType.DMA,
        pltpu.SemaphoreType.DMA,
        pltpu.SemaphoreType.DMA,
        pltpu.SemaphoreType.DMA,
        pltpu.SemaphoreType.DMA,
        pltpu.SemaphoreType.DMA,
    ],
)
def _deg_kernel(idx_hbm, out_hbm, x0, x1, x2, x3, acc, onesv, zv,
                i0, i1, i2, i3, d0, d1, d2, d3):
    c = lax.axis_index("c")
    s = lax.axis_index("s")
    ixs = (x0, x1, x2, x3)
    isems = (i0, i1, i2, i3)
    dsems = (d0, d1, d2, d3)
    NB4 = 4 * NBLK

    def zi(i, _):
        zv[pl.ds(i * 16, 16)] = jnp.zeros((16,), jnp.float32)
        return 0

    lax.fori_loop(0, ROWS_PT // 16, zi, 0)
    for j in range(BB // 16):
        onesv[pl.ds(j * 16, 16)] = jnp.ones((16,), jnp.float32)

    def off_of(t):
        k = t // NBLK
        bb = t - k * NBLK
        r = c + 2 * (k // 2)
        side = k - 2 * (k // 2)
        return (r * 2 + side) * E + s * TPB + bb * BB

    def load_idx(t, slot):
        pltpu.async_copy(idx_hbm.at[pl.ds(off_of(t), BB)], ixs[slot],
                         isems[slot])

    def wait_idx_sem(dst, sem):
        pltpu.make_async_copy(idx_hbm.at[pl.ds(0, BB)], dst, sem).wait()

    load_idx(0, 0)
    load_idx(1, 1)
    for z in range(4):
        pltpu.sync_copy(zv, acc.at[pl.ds((s * 4 + z) * ROWS_PT, ROWS_PT)])
    plsc.subcore_barrier()

    def grp(g, _):
        for sub in range(4):
            t = 4 * g + sub
            nslot = (sub + 2) % 4

            @pl.when(t + 2 < NB4)
            def _prefetch():
                @pl.when(t >= 2)
                def _reuse():
                    wait_idx_sem(ixs[nslot], dsems[nslot])
                load_idx(t + 2, nslot)

            wait_idx_sem(ixs[sub], isems[sub])
            pltpu.async_copy(onesv, acc.at[ixs[sub]], dsems[sub], add=True)
        return 0

    lax.fori_loop(0, NB4 // 4, grp, 0)
    for sub in range(4):
        wait_idx_sem(ixs[(2 + sub) % 4], dsems[(2 + sub) % 4])
    plsc.subcore_barrier()
    for k in range(4):
        r = c + 2 * (k // 2)
        side = k % 2
        pltpu.sync_copy(
            acc.at[pl.ds(k * NP + s * ROWS_PT, ROWS_PT)],
            out_hbm.at[pl.ds((r * 2 + side) * NP + s * ROWS_PT, ROWS_PT)],
        )


@functools.partial(
    pl.kernel,
    out_type=jax.ShapeDtypeStruct((R * NP, D), jnp.float32),
    mesh=_mesh(),
    scratch_types=[
        pltpu.VMEM_SHARED((NP, D), jnp.float32),
        pltpu.VMEM((BB,), jnp.int32),
        pltpu.VMEM((BB,), jnp.int32),
        pltpu.VMEM((BB,), jnp.int32),
        pltpu.VMEM((BB,), jnp.int32),
        pltpu.VMEM((BB, D), jnp.float32),
        pltpu.VMEM((BB, D), jnp.float32),
        pltpu.VMEM((BB, D), jnp.float32),
        pltpu.VMEM((BB, D), jnp.float32),
        pltpu.VMEM((16, D), jnp.float32),
        pltpu.SemaphoreType.DMA,
        pltpu.SemaphoreType.DMA,
        pltpu.SemaphoreType.DMA,
        pltpu.SemaphoreType.DMA,
        pltpu.SemaphoreType.DMA,
        pltpu.SemaphoreType.DMA,
        pltpu.SemaphoreType.DMA,
        pltpu.SemaphoreType.DMA,
        pltpu.SemaphoreType.DMA,
        pltpu.SemaphoreType.DMA,
        pltpu.SemaphoreType.DMA,
    ],
)
def _edge_kernel(hs_hbm, rw_hbm, sidx_hbm, didx_hbm, agg_hbm,
                 acc, si0, si1, di0, di1, hrow0, hrow1, rrow0, rrow1, zbuf,
                 i0, i1, j0, j1, g0, g1, r0, r1, s0, s1, zsem):
    c = lax.axis_index("c")
    s = lax.axis_index("s")
    sidxs = (si0, si1)
    didxs = (di0, di1)
    hrows = (hrow0, hrow1)
    rrows = (rrow0, rrow1)
    isems = (i0, i1)
    jsems = (j0, j1)
    gsems = (g0, g1)
    rsems = (r0, r1)
    ssems = (s0, s1)

    def zi(i, _):
        for j in range(D // 16):
            zbuf[i, pl.ds(j * 16, 16)] = jnp.zeros((16,), jnp.float32)
        return 0

    lax.fori_loop(0, 16, zi, 0)

    def wait_idx(dst, sem):
        pltpu.make_async_copy(sidx_hbm.at[pl.ds(0, BB)], dst, sem).wait()

    def wait_rows(dst, sem):
        pltpu.make_async_copy(hs_hbm.at[pl.ds(0, BB)], dst, sem).wait()

    for rel_i in range(2):
        r = c + 2 * rel_i
        eoff = r * E + s * TPB

        def load_sidx(b, slot):
            pltpu.async_copy(sidx_hbm.at[pl.ds(eoff + b * BB, BB)],
                             sidxs[slot], isems[slot])

        def load_didx(b, slot):
            pltpu.async_copy(didx_hbm.at[pl.ds(eoff + b * BB, BB)],
                             didxs[slot], jsems[slot])

        def load_rw(b, slot):
            pltpu.async_copy(rw_hbm.at[pl.ds(eoff + b * BB, BB)],
                             rrows[slot], rsems[slot])

        def zz(z, _):
            pltpu.async_copy(zbuf, acc.at[pl.ds(s * ROWS_PT + z * 16, 16)],
                             zsem)
            return 0

        load_sidx(0, 0)
        load_didx(0, 0)
        load_rw(0, 0)
        lax.fori_loop(0, ROWS_PT // 16, zz, 0)

        def zw(z, _):
            pltpu.make_async_copy(hs_hbm.at[pl.ds(0, 16)], zbuf, zsem).wait()
            return 0

        lax.fori_loop(0, ROWS_PT // 16, zw, 0)
        plsc.subcore_barrier()
        wait_idx(sidxs[0], isems[0])
        pltpu.async_copy(hs_hbm.at[sidxs[0]], hrows[0], gsems[0])
        load_sidx(1, 1)

        def step(b, slot, other, first, has_next, has_next2):
            if not first:
                wait_rows(hrows[other], ssems[other])
            wait_rows(hrows[slot], gsems[slot])
            if has_next:
                load_didx(b + 1, other)
                load_rw(b + 1, other)
                wait_idx(sidxs[other], isems[other])
                pltpu.async_copy(hs_hbm.at[sidxs[other]], hrows[other],
                                 gsems[other])
            if has_next2:
                load_sidx(b + 2, slot)
            wait_rows(rrows[slot], rsems[slot])

            def mrow(i, _):
                for j in range(D // 16):
                    hrows[slot][i, pl.ds(j * 16, 16)] = (
                        hrows[slot][i, pl.ds(j * 16, 16)]
                        * rrows[slot][i, pl.ds(j * 16, 16)]
                    )
                return 0

            lax.fori_loop(0, BB, mrow, 0)
            wait_idx(didxs[slot], jsems[slot])
            pltpu.async_copy(hrows[slot], acc.at[didxs[slot]],
                             ssems[slot], add=True)

        step(0, 0, 1, True, True, True)

        def pair(p, _):
            step(2 * p + 1, 1, 0, False, True, True)
            step(2 * p + 2, 0, 1, False, True, True)
            return 0

        lax.fori_loop(0, (NBLK - 3) // 2, pair, 0)
        step(NBLK - 2, 1, 0, False, True, False)
        step(NBLK - 1, 0, 1, False, False, False)
        wait_rows(hrows[0], ssems[0])
        plsc.subcore_barrier()
        pltpu.sync_copy(
            acc.at[pl.ds(s * ROWS_PT, ROWS_PT)],
            agg_hbm.at[pl.ds(r * NP + s * ROWS_PT, ROWS_PT)],
        )


def _matmul_bias_body(x_ref, w_ref, b_ref, o_ref):
    o_ref[...] = (
        jnp.dot(x_ref[...], w_ref[...], preferred_element_type=jnp.float32)
        + b_ref[...]
    )


def _rw_call(ews_flat, wT, b):
    BA = 2560
    return pl.pallas_call(
        _matmul_bias_body,
        grid=(R * E // BA,),
        in_specs=[
            pl.BlockSpec((BA, D), lambda i: (i, 0)),
            pl.BlockSpec((D, D), lambda i: (0, 0)),
            pl.BlockSpec((1, D), lambda i: (0, 0)),
        ],
        out_specs=pl.BlockSpec((BA, D), lambda i: (i, 0)),
        out_shape=jax.ShapeDtypeStruct((R * E, D), jnp.float32),
    )(ews_flat, wT, b)


def _h0_call(feat, wT, b):
    return pl.pallas_call(
        _matmul_bias_body,
        out_shape=jax.ShapeDtypeStruct((N, D), jnp.float32),
    )(feat, wT, b)


def _prep_body(h0_ref, dt_ref, sc_ref, hs_ref):
    sc = lax.rsqrt(jnp.maximum(dt_ref[...], 1.0))
    sc_ref[...] = sc
    h = h0_ref[...]
    for r in range(R):
        hs_ref[r, 0:N, :] = h * sc[0:N, 2 * r:2 * r + 1]


def _prep_call(h0, degs_t):
    return pl.pallas_call(
        _prep_body,
        out_shape=[
            jax.ShapeDtypeStruct((NP, 8), jnp.float32),
            jax.ShapeDtypeStruct((R, NP, D), jnp.float32),
        ],
    )(h0, degs_t)


_BN = 1000


def _combine_emb(agg_ref, w_ref, b_ref, sc_ref):
    sc = sc_ref[...]
    acc = jnp.zeros((_BN, D), jnp.float32)
    for r in range(R):
        acc = acc + (
            jnp.dot(agg_ref[r], w_ref[r], preferred_element_type=jnp.float32)
            * sc[:, 2 * r + 1:2 * r + 2]
        )
    return acc * (1.0 / R) + jnp.sum(b_ref[...], axis=0, keepdims=True) * (1.0 / R)


def _combine_mid_body(agg_ref, w_ref, b_ref, sc_ref, hs2_ref):
    emb = _combine_emb(agg_ref, w_ref, b_ref, sc_ref)
    sc = sc_ref[...]
    for r in range(R):
        hs2_ref[r] = emb * sc[:, 2 * r:2 * r + 1]


_COMBINE_IN_SPECS = [
    pl.BlockSpec((R, _BN, D), lambda i: (0, i, 0)),
    pl.BlockSpec((R, D, D), lambda i: (0, 0, 0)),
    pl.BlockSpec((R, D), lambda i: (0, 0)),
    pl.BlockSpec((_BN, 8), lambda i: (i, 0)),
]


def _combine_mid_call(agg3, W, b, scales):
    return pl.pallas_call(
        _combine_mid_body,
        grid=(N // _BN,),
        in_specs=_COMBINE_IN_SPECS,
        out_specs=pl.BlockSpec((R, _BN, D), lambda i: (0, i, 0)),
        out_shape=jax.ShapeDtypeStruct((R, NP, D), jnp.float32),
    )(agg3, W, b, scales)


def _combine_final_body(agg_ref, w_ref, b_ref, sc_ref, out_ref):
    out_ref[...] = _combine_emb(agg_ref, w_ref, b_ref, sc_ref)


def _combine_final_call(agg3, W, b, scales):
    return pl.pallas_call(
        _combine_final_body,
        grid=(N // _BN,),
        in_specs=_COMBINE_IN_SPECS,
        out_specs=pl.BlockSpec((_BN, D), lambda i: (i, 0)),
        out_shape=jax.ShapeDtypeStruct((N, D), jnp.float32),
    )(agg3, W, b, scales)


def kernel(feat, edge_index_r0, edge_index_r1, edge_index_r2, edge_index_r3,
           edge_weight_r0, edge_weight_r1, edge_weight_r2, edge_weight_r3,
           node_fc_W, node_fc_b, rela_fc_W, rela_fc_b, W1, b1, W2, b2):
    eis = [edge_index_r0, edge_index_r1, edge_index_r2, edge_index_r3]
    ews = [edge_weight_r0, edge_weight_r1, edge_weight_r2, edge_weight_r3]

    srcs = [eis[r][0].astype(jnp.int32) for r in range(R)]
    dsts = [eis[r][1].astype(jnp.int32) for r in range(R)]
    src_adj = jnp.concatenate([srcs[r] + r * NP for r in range(R)])
    dst_flat = jnp.concatenate(dsts)
    deg_idx = jnp.concatenate(
        [jnp.concatenate([srcs[r] + ((r // 2) * 2) * NP,
                          dsts[r] + ((r // 2) * 2 + 1) * NP])
         for r in range(R)])
    ews_flat = jnp.concatenate(ews, axis=0)

    rw_flat = _rw_call(ews_flat, rela_fc_W.T, rela_fc_b[None, :])
    h0 = _h0_call(feat, node_fc_W.T, node_fc_b[None, :])
    degs_flat = _deg_kernel(deg_idx)
    degs_t = degs_flat.reshape(R * 2, NP).T
    scales, hs1 = _prep_call(h0, degs_t)

    agg1 = _edge_kernel(hs1.reshape(R * NP, D), rw_flat, src_adj, dst_flat)
    hs2 = _combine_mid_call(agg1.reshape(R, NP, D), W1, b1, scales)
    agg2 = _edge_kernel(hs2.reshape(R * NP, D), rw_flat, src_adj, dst_flat)
    return _combine_final_call(agg2.reshape(R, NP, D), W2, b2, scales)


# speedup vs baseline: 5.2340x; 1.3187x over previous
import functools

import jax
import jax.numpy as jnp
from jax import lax
from jax.experimental import pallas as pl
from jax.experimental.pallas import tpu as pltpu
from jax.experimental.pallas import tpu_sc as plsc

N = 10000
D = 128
R = 4
E = 160000
NP = 10240
NS = 16
NC = 2
TPB = E // NS
BB = 80
NBLK = TPB // BB
ROWS_PT = NP // NS


def _mesh():
    return plsc.VectorSubcoreMesh(core_axis_name="c", subcore_axis_name="s")


@functools.partial(
    pl.kernel,
    out_type=jax.ShapeDtypeStruct((R * 2 * NP,), jnp.float32),
    mesh=_mesh(),
    scratch_types=[
        pltpu.VMEM((BB,), jnp.int32),
        pltpu.VMEM((BB,), jnp.int32),
        pltpu.VMEM((BB,), jnp.int32),
        pltpu.VMEM((BB,), jnp.int32),
        pltpu.VMEM_SHARED((4 * NP,), jnp.float32),
        pltpu.VMEM((BB,), jnp.float32),
        pltpu.VMEM((ROWS_PT,), jnp.float32),
        pltpu.SemaphoreType.DMA,
        pltpu.SemaphoreType.DMA,
        pltpu.SemaphoreType.DMA,
        pltpu.SemaphoreType.DMA,
        pltpu.SemaphoreType.DMA,
        pltpu.SemaphoreType.DMA,
        pltpu.SemaphoreType.DMA,
        pltpu.SemaphoreType.DMA,
    ],
)
def _deg_kernel(idx_hbm, out_hbm, x0, x1, x2, x3, acc, onesv, zv,
                i0, i1, i2, i3, d0, d1, d2, d3):
    c = lax.axis_index("c")
    s = lax.axis_index("s")
    ixs = (x0, x1, x2, x3)
    isems = (i0, i1, i2, i3)
    dsems = (d0, d1, d2, d3)
    NB4 = 4 * NBLK

    def zi(i, _):
        zv[pl.ds(i * 16, 16)] = jnp.zeros((16,), jnp.float32)
        return 0

    lax.fori_loop(0, ROWS_PT // 16, zi, 0)
    for j in range(BB // 16):
        onesv[pl.ds(j * 16, 16)] = jnp.ones((16,), jnp.float32)

    def off_of(t):
        k = t // NBLK
        bb = t - k * NBLK
        r = c + 2 * (k // 2)
        side = k - 2 * (k // 2)
        return (r * 2 + side) * E + s * TPB + bb * BB

    def load_idx(t, slot):
        pltpu.async_copy(idx_hbm.at[pl.ds(off_of(t), BB)], ixs[slot],
                         isems[slot])

    def wait_idx_sem(dst, sem):
        pltpu.make_async_copy(idx_hbm.at[pl.ds(0, BB)], dst, sem).wait()

    load_idx(0, 0)
    load_idx(1, 1)
    for z in range(4):
        pltpu.sync_copy(zv, acc.at[pl.ds((s * 4 + z) * ROWS_PT, ROWS_PT)])
    plsc.subcore_barrier()

    def grp(g, _):
        for sub in range(4):
            t = 4 * g + sub
            nslot = (sub + 2) % 4

            @pl.when(t + 2 < NB4)
            def _prefetch():
                @pl.when(t >= 2)
                def _reuse():
                    wait_idx_sem(ixs[nslot], dsems[nslot])
                load_idx(t + 2, nslot)

            wait_idx_sem(ixs[sub], isems[sub])
            pltpu.async_copy(onesv, acc.at[ixs[sub]], dsems[sub], add=True)
        return 0

    lax.fori_loop(0, NB4 // 4, grp, 0)
    for sub in range(4):
        wait_idx_sem(ixs[(2 + sub) % 4], dsems[(2 + sub) % 4])
    plsc.subcore_barrier()
    for k in range(4):
        r = c + 2 * (k // 2)
        side = k % 2
        pltpu.sync_copy(
            acc.at[pl.ds(k * NP + s * ROWS_PT, ROWS_PT)],
            out_hbm.at[pl.ds((r * 2 + side) * NP + s * ROWS_PT, ROWS_PT)],
        )


@functools.partial(
    pl.kernel,
    out_type=jax.ShapeDtypeStruct((R * NP, D), jnp.float32),
    mesh=_mesh(),
    scratch_types=[
        pltpu.VMEM_SHARED((NP, D), jnp.float32),
        pltpu.VMEM((BB,), jnp.int32),
        pltpu.VMEM((BB,), jnp.int32),
        pltpu.VMEM((BB,), jnp.int32),
        pltpu.VMEM((BB,), jnp.int32),
        pltpu.VMEM((BB, D), jnp.float32),
        pltpu.VMEM((BB, D), jnp.float32),
        pltpu.VMEM((BB, D), jnp.float32),
        pltpu.VMEM((BB, D), jnp.float32),
        pltpu.VMEM((16, D), jnp.float32),
        pltpu.SemaphoreType.DMA,
        pltpu.SemaphoreType.DMA,
        pltpu.SemaphoreType.DMA,
        pltpu.SemaphoreType.DMA,
        pltpu.SemaphoreType.DMA,
        pltpu.SemaphoreType.DMA,
        pltpu.SemaphoreType.DMA,
        pltpu.SemaphoreType.DMA,
        pltpu.SemaphoreType.DMA,
        pltpu.SemaphoreType.DMA,
        pltpu.SemaphoreType.DMA,
    ],
)
def _edge_kernel(hs_hbm, rw0_hbm, rw1_hbm, rw2_hbm, rw3_hbm,
                 sidx_hbm, didx_hbm, agg_hbm,
                 acc, si0, si1, di0, di1, hrow0, hrow1, rrow0, rrow1, zbuf,
                 i0, i1, j0, j1, g0, g1, r0, r1, s0, s1, zsem):
    rw_hbms = (rw0_hbm, rw1_hbm, rw2_hbm, rw3_hbm)
    c = lax.axis_index("c")
    s = lax.axis_index("s")
    sidxs = (si0, si1)
    didxs = (di0, di1)
    hrows = (hrow0, hrow1)
    rrows = (rrow0, rrow1)
    isems = (i0, i1)
    jsems = (j0, j1)
    gsems = (g0, g1)
    rsems = (r0, r1)
    ssems = (s0, s1)

    def zi(i, _):
        for j in range(D // 16):
            zbuf[i, pl.ds(j * 16, 16)] = jnp.zeros((16,), jnp.float32)
        return 0

    lax.fori_loop(0, 16, zi, 0)

    def wait_idx(dst, sem):
        pltpu.make_async_copy(sidx_hbm.at[pl.ds(0, BB)], dst, sem).wait()

    def wait_rows(dst, sem):
        pltpu.make_async_copy(hs_hbm.at[pl.ds(0, BB)], dst, sem).wait()

    for rel_i in range(2):
        r = c + 2 * rel_i
        eoff = r * E + s * TPB

        def load_sidx(b, slot):
            pltpu.async_copy(sidx_hbm.at[pl.ds(eoff + b * BB, BB)],
                             sidxs[slot], isems[slot])

        def load_didx(b, slot):
            pltpu.async_copy(didx_hbm.at[pl.ds(eoff + b * BB, BB)],
                             didxs[slot], jsems[slot])

        def load_rw(b, slot):
            for rr in (2 * rel_i, 2 * rel_i + 1):
                @pl.when(r == rr)
                def _ld(rr=rr):
                    pltpu.async_copy(
                        rw_hbms[rr].at[pl.ds(s * TPB + b * BB, BB)],
                        rrows[slot], rsems[slot])

        def zz(z, _):
            pltpu.async_copy(zbuf, acc.at[pl.ds(s * ROWS_PT + z * 16, 16)],
                             zsem)
            return 0

        load_sidx(0, 0)
        load_didx(0, 0)
        load_rw(0, 0)
        lax.fori_loop(0, ROWS_PT // 16, zz, 0)

        def zw(z, _):
            pltpu.make_async_copy(hs_hbm.at[pl.ds(0, 16)], zbuf, zsem).wait()
            return 0

        lax.fori_loop(0, ROWS_PT // 16, zw, 0)
        plsc.subcore_barrier()
        wait_idx(sidxs[0], isems[0])
        pltpu.async_copy(hs_hbm.at[sidxs[0]], hrows[0], gsems[0])
        load_sidx(1, 1)

        def step(b, slot, other, first, has_next, has_next2):
            if not first:
                wait_rows(hrows[other], ssems[other])
            wait_rows(hrows[slot], gsems[slot])
            if has_next:
                load_didx(b + 1, other)
                load_rw(b + 1, other)
                wait_idx(sidxs[other], isems[other])
                pltpu.async_copy(hs_hbm.at[sidxs[other]], hrows[other],
                                 gsems[other])
            if has_next2:
                load_sidx(b + 2, slot)
            wait_rows(rrows[slot], rsems[slot])

            def mrow(i, _):
                for j in range(D // 16):
                    hrows[slot][i, pl.ds(j * 16, 16)] = (
                        hrows[slot][i, pl.ds(j * 16, 16)]
                        * rrows[slot][i, pl.ds(j * 16, 16)]
                    )
                return 0

            lax.fori_loop(0, BB, mrow, 0)
            wait_idx(didxs[slot], jsems[slot])
            pltpu.async_copy(hrows[slot], acc.at[didxs[slot]],
                             ssems[slot], add=True)

        step(0, 0, 1, True, True, True)

        def pair(p, _):
            step(2 * p + 1, 1, 0, False, True, True)
            step(2 * p + 2, 0, 1, False, True, True)
            return 0

        lax.fori_loop(0, (NBLK - 3) // 2, pair, 0)
        step(NBLK - 2, 1, 0, False, True, False)
        step(NBLK - 1, 0, 1, False, False, False)
        wait_rows(hrows[0], ssems[0])
        plsc.subcore_barrier()
        pltpu.sync_copy(
            acc.at[pl.ds(s * ROWS_PT, ROWS_PT)],
            agg_hbm.at[pl.ds(r * NP + s * ROWS_PT, ROWS_PT)],
        )


def _matmul_bias_body(x_ref, w_ref, b_ref, o_ref):
    o_ref[...] = (
        jnp.dot(x_ref[...], w_ref[...], preferred_element_type=jnp.float32)
        + b_ref[...]
    )


def _rw_body(x0, x1, x2, x3, w_ref, b_ref, o0, o1, o2, o3):
    for x_ref, o_ref in ((x0, o0), (x1, o1), (x2, o2), (x3, o3)):
        o_ref[...] = (
            jnp.dot(x_ref[...], w_ref[...], preferred_element_type=jnp.float32)
            + b_ref[...]
        )


def _rw_call(ews, wT, b):
    BA = 2000
    blk = pl.BlockSpec((BA, D), lambda i: (i, 0))
    return pl.pallas_call(
        _rw_body,
        grid=(E // BA,),
        in_specs=[blk, blk, blk, blk,
                  pl.BlockSpec((D, D), lambda i: (0, 0)),
                  pl.BlockSpec((1, D), lambda i: (0, 0))],
        out_specs=[blk, blk, blk, blk],
        out_shape=[jax.ShapeDtypeStruct((E, D), jnp.float32)] * 4,
    )(*ews, wT, b)


def _h0_call(feat, wT, b):
    return pl.pallas_call(
        _matmul_bias_body,
        out_shape=jax.ShapeDtypeStruct((N, D), jnp.float32),
    )(feat, wT, b)


def _prep_body(h0_ref, dt_ref, sc_ref, hs_ref):
    sc = lax.rsqrt(jnp.maximum(dt_ref[...], 1.0))
    sc_ref[...] = sc
    h = h0_ref[...]
    for r in range(R):
        hs_ref[r, 0:N, :] = h * sc[0:N, 2 * r:2 * r + 1]


def _prep_call(h0, degs_t):
    return pl.pallas_call(
        _prep_body,
        out_shape=[
            jax.ShapeDtypeStruct((NP, 8), jnp.float32),
            jax.ShapeDtypeStruct((R, NP, D), jnp.float32),
        ],
    )(h0, degs_t)


_BN = 1000


def _combine_emb(agg_ref, w_ref, b_ref, sc_ref):
    sc = sc_ref[...]
    acc = jnp.zeros((_BN, D), jnp.float32)
    for r in range(R):
        acc = acc + (
            jnp.dot(agg_ref[r], w_ref[r], preferred_element_type=jnp.float32)
            * sc[:, 2 * r + 1:2 * r + 2]
        )
    return acc * (1.0 / R) + jnp.sum(b_ref[...], axis=0, keepdims=True) * (1.0 / R)


def _combine_mid_body(agg_ref, w_ref, b_ref, sc_ref, hs2_ref):
    emb = _combine_emb(agg_ref, w_ref, b_ref, sc_ref)
    sc = sc_ref[...]
    for r in range(R):
        hs2_ref[r] = emb * sc[:, 2 * r:2 * r + 1]


_COMBINE_IN_SPECS = [
    pl.BlockSpec((R, _BN, D), lambda i: (0, i, 0)),
    pl.BlockSpec((R, D, D), lambda i: (0, 0, 0)),
    pl.BlockSpec((R, D), lambda i: (0, 0)),
    pl.BlockSpec((_BN, 8), lambda i: (i, 0)),
]


def _combine_mid_call(agg3, W, b, scales):
    return pl.pallas_call(
        _combine_mid_body,
        grid=(N // _BN,),
        in_specs=_COMBINE_IN_SPECS,
        out_specs=pl.BlockSpec((R, _BN, D), lambda i: (0, i, 0)),
        out_shape=jax.ShapeDtypeStruct((R, NP, D), jnp.float32),
    )(agg3, W, b, scales)


def _combine_final_body(agg_ref, w_ref, b_ref, sc_ref, out_ref):
    out_ref[...] = _combine_emb(agg_ref, w_ref, b_ref, sc_ref)


def _combine_final_call(agg3, W, b, scales):
    return pl.pallas_call(
        _combine_final_body,
        grid=(N // _BN,),
        in_specs=_COMBINE_IN_SPECS,
        out_specs=pl.BlockSpec((_BN, D), lambda i: (i, 0)),
        out_shape=jax.ShapeDtypeStruct((N, D), jnp.float32),
    )(agg3, W, b, scales)


def kernel(feat, edge_index_r0, edge_index_r1, edge_index_r2, edge_index_r3,
           edge_weight_r0, edge_weight_r1, edge_weight_r2, edge_weight_r3,
           node_fc_W, node_fc_b, rela_fc_W, rela_fc_b, W1, b1, W2, b2):
    eis = [edge_index_r0, edge_index_r1, edge_index_r2, edge_index_r3]
    ews = [edge_weight_r0, edge_weight_r1, edge_weight_r2, edge_weight_r3]

    srcs = [eis[r][0].astype(jnp.int32) for r in range(R)]
    dsts = [eis[r][1].astype(jnp.int32) for r in range(R)]
    src_adj = jnp.concatenate([srcs[r] + r * NP for r in range(R)])
    dst_flat = jnp.concatenate(dsts)
    deg_idx = jnp.concatenate(
        [jnp.concatenate([srcs[r] + ((r // 2) * 2) * NP,
                          dsts[r] + ((r // 2) * 2 + 1) * NP])
         for r in range(R)])
    rw0, rw1, rw2, rw3 = _rw_call(ews, rela_fc_W.T, rela_fc_b[None, :])
    h0 = _h0_call(feat, node_fc_W.T, node_fc_b[None, :])
    degs_flat = _deg_kernel(deg_idx)
    degs_t = degs_flat.reshape(R * 2, NP).T
    scales, hs1 = _prep_call(h0, degs_t)

    agg1 = _edge_kernel(hs1.reshape(R * NP, D), rw0, rw1, rw2, rw3,
                        src_adj, dst_flat)
    hs2 = _combine_mid_call(agg1.reshape(R, NP, D), W1, b1, scales)
    agg2 = _edge_kernel(hs2.reshape(R * NP, D), rw0, rw1, rw2, rw3,
                        src_adj, dst_flat)
    return _combine_final_call(agg2.reshape(R, NP, D), W2, b2, scales)


# speedup vs baseline: 5.4930x; 1.0495x over previous
import functools

import jax
import jax.numpy as jnp
from jax import lax
from jax.experimental import pallas as pl
from jax.experimental.pallas import tpu as pltpu
from jax.experimental.pallas import tpu_sc as plsc

N = 10000
D = 128
R = 4
E = 160000
NP = 10240
NS = 16
NC = 2
TPB = E // NS
BB = 80
NBLK = TPB // BB
ROWS_PT = NP // NS


def _mesh():
    return plsc.VectorSubcoreMesh(core_axis_name="c", subcore_axis_name="s")


@functools.partial(
    pl.kernel,
    out_type=jax.ShapeDtypeStruct((R * 2 * NP,), jnp.float32),
    mesh=_mesh(),
    scratch_types=[
        pltpu.VMEM((BB,), jnp.int32),
        pltpu.VMEM((BB,), jnp.int32),
        pltpu.VMEM((BB,), jnp.int32),
        pltpu.VMEM((BB,), jnp.int32),
        pltpu.VMEM_SHARED((4 * NP,), jnp.float32),
        pltpu.VMEM((BB,), jnp.float32),
        pltpu.VMEM((ROWS_PT,), jnp.float32),
        pltpu.SemaphoreType.DMA,
        pltpu.SemaphoreType.DMA,
        pltpu.SemaphoreType.DMA,
        pltpu.SemaphoreType.DMA,
        pltpu.SemaphoreType.DMA,
        pltpu.SemaphoreType.DMA,
        pltpu.SemaphoreType.DMA,
        pltpu.SemaphoreType.DMA,
    ],
)
def _deg_kernel(idx_hbm, out_hbm, x0, x1, x2, x3, acc, onesv, zv,
                i0, i1, i2, i3, d0, d1, d2, d3):
    c = lax.axis_index("c")
    s = lax.axis_index("s")
    ixs = (x0, x1, x2, x3)
    isems = (i0, i1, i2, i3)
    dsems = (d0, d1, d2, d3)
    NB4 = 4 * NBLK

    def zi(i, _):
        zv[pl.ds(i * 16, 16)] = jnp.zeros((16,), jnp.float32)
        return 0

    lax.fori_loop(0, ROWS_PT // 16, zi, 0)
    for j in range(BB // 16):
        onesv[pl.ds(j * 16, 16)] = jnp.ones((16,), jnp.float32)

    def off_of(t):
        k = t // NBLK
        bb = t - k * NBLK
        r = c + 2 * (k // 2)
        side = k - 2 * (k // 2)
        return (r * 2 + side) * E + s * TPB + bb * BB

    def load_idx(t, slot):
        pltpu.async_copy(idx_hbm.at[pl.ds(off_of(t), BB)], ixs[slot],
                         isems[slot])

    def wait_idx_sem(dst, sem):
        pltpu.make_async_copy(idx_hbm.at[pl.ds(0, BB)], dst, sem).wait()

    load_idx(0, 0)
    load_idx(1, 1)
    for z in range(4):
        pltpu.sync_copy(zv, acc.at[pl.ds((s * 4 + z) * ROWS_PT, ROWS_PT)])
    plsc.subcore_barrier()

    def grp(g, _):
        for sub in range(4):
            t = 4 * g + sub
            nslot = (sub + 2) % 4

            @pl.when(t + 2 < NB4)
            def _prefetch():
                @pl.when(t >= 2)
                def _reuse():
                    wait_idx_sem(ixs[nslot], dsems[nslot])
                load_idx(t + 2, nslot)

            wait_idx_sem(ixs[sub], isems[sub])
            pltpu.async_copy(onesv, acc.at[ixs[sub]], dsems[sub], add=True)
        return 0

    lax.fori_loop(0, NB4 // 4, grp, 0)
    for sub in range(4):
        wait_idx_sem(ixs[(2 + sub) % 4], dsems[(2 + sub) % 4])
    plsc.subcore_barrier()
    for k in range(4):
        r = c + 2 * (k // 2)
        side = k % 2
        pltpu.sync_copy(
            acc.at[pl.ds(k * NP + s * ROWS_PT, ROWS_PT)],
            out_hbm.at[pl.ds((r * 2 + side) * NP + s * ROWS_PT, ROWS_PT)],
        )


@functools.partial(
    pl.kernel,
    out_type=jax.ShapeDtypeStruct((R * NP, D), jnp.float32),
    mesh=_mesh(),
    scratch_types=[
        pltpu.VMEM_SHARED((NP, D), jnp.float32),
        pltpu.VMEM((BB,), jnp.int32),
        pltpu.VMEM((BB,), jnp.int32),
        pltpu.VMEM((BB,), jnp.int32),
        pltpu.VMEM((BB,), jnp.int32),
        pltpu.VMEM((BB, D), jnp.float32),
        pltpu.VMEM((BB, D), jnp.float32),
        pltpu.VMEM((BB, D), jnp.float32),
        pltpu.VMEM((BB, D), jnp.float32),
        pltpu.VMEM((16, D), jnp.float32),
        pltpu.SemaphoreType.DMA,
        pltpu.SemaphoreType.DMA,
        pltpu.SemaphoreType.DMA,
        pltpu.SemaphoreType.DMA,
        pltpu.SemaphoreType.DMA,
        pltpu.SemaphoreType.DMA,
        pltpu.SemaphoreType.DMA,
        pltpu.SemaphoreType.DMA,
        pltpu.SemaphoreType.DMA,
        pltpu.SemaphoreType.DMA,
        pltpu.SemaphoreType.DMA,
    ],
)
def _edge_kernel(hs_hbm, rw0_hbm, rw1_hbm, rw2_hbm, rw3_hbm,
                 sidx_hbm, didx_hbm, agg_hbm,
                 acc, si0, si1, di0, di1, hrow0, hrow1, rrow0, rrow1, zbuf,
                 i0, i1, j0, j1, g0, g1, r0, r1, s0, s1, zsem):
    rw_hbms = (rw0_hbm, rw1_hbm, rw2_hbm, rw3_hbm)
    c = lax.axis_index("c")
    s = lax.axis_index("s")
    sidxs = (si0, si1)
    didxs = (di0, di1)
    hrows = (hrow0, hrow1)
    rrows = (rrow0, rrow1)
    isems = (i0, i1)
    jsems = (j0, j1)
    gsems = (g0, g1)
    rsems = (r0, r1)
    ssems = (s0, s1)

    def zi(i, _):
        for j in range(D // 16):
            zbuf[i, pl.ds(j * 16, 16)] = jnp.zeros((16,), jnp.float32)
        return 0

    lax.fori_loop(0, 16, zi, 0)

    def wait_idx(dst, sem):
        pltpu.make_async_copy(sidx_hbm.at[pl.ds(0, BB)], dst, sem).wait()

    def wait_rows(dst, sem):
        pltpu.make_async_copy(hs_hbm.at[pl.ds(0, BB)], dst, sem).wait()

    for rel_i in range(2):
        r = c + 2 * rel_i
        eoff = r * E + s * TPB

        def load_sidx(b, slot):
            pltpu.async_copy(sidx_hbm.at[pl.ds(eoff + b * BB, BB)],
                             sidxs[slot], isems[slot])

        def load_didx(b, slot):
            pltpu.async_copy(didx_hbm.at[pl.ds(eoff + b * BB, BB)],
                             didxs[slot], jsems[slot])

        def load_rw(b, slot):
            for rr in (2 * rel_i, 2 * rel_i + 1):
                @pl.when(r == rr)
                def _ld(rr=rr):
                    pltpu.async_copy(
                        rw_hbms[rr].at[pl.ds(s * TPB + b * BB, BB)],
                        rrows[slot], rsems[slot])

        def zz(z, _):
            pltpu.async_copy(zbuf, acc.at[pl.ds(s * ROWS_PT + z * 16, 16)],
                             zsem)
            return 0

        load_sidx(0, 0)
        load_didx(0, 0)
        load_rw(0, 0)
        lax.fori_loop(0, ROWS_PT // 16, zz, 0)

        def zw(z, _):
            pltpu.make_async_copy(hs_hbm.at[pl.ds(0, 16)], zbuf, zsem).wait()
            return 0

        lax.fori_loop(0, ROWS_PT // 16, zw, 0)
        plsc.subcore_barrier()
        wait_idx(sidxs[0], isems[0])
        pltpu.async_copy(hs_hbm.at[sidxs[0]], hrows[0], gsems[0])
        load_sidx(1, 1)

        def step(b, slot, other, first, has_next, has_next2):
            if not first:
                wait_rows(hrows[other], ssems[other])
            if has_next:
                load_didx(b + 1, other)
                load_rw(b + 1, other)
                wait_idx(sidxs[other], isems[other])
                pltpu.async_copy(hs_hbm.at[sidxs[other]], hrows[other],
                                 gsems[other])
            wait_rows(hrows[slot], gsems[slot])
            if has_next2:
                load_sidx(b + 2, slot)
            wait_rows(rrows[slot], rsems[slot])

            def mrow(i, _):
                for j in range(D // 16):
                    hrows[slot][i, pl.ds(j * 16, 16)] = (
                        hrows[slot][i, pl.ds(j * 16, 16)]
                        * rrows[slot][i, pl.ds(j * 16, 16)]
                    )
                return 0

            lax.fori_loop(0, BB, mrow, 0)
            wait_idx(didxs[slot], jsems[slot])
            pltpu.async_copy(hrows[slot], acc.at[didxs[slot]],
                             ssems[slot], add=True)

        step(0, 0, 1, True, True, True)

        def pair(p, _):
            step(2 * p + 1, 1, 0, False, True, True)
            step(2 * p + 2, 0, 1, False, True, True)
            return 0

        lax.fori_loop(0, (NBLK - 3) // 2, pair, 0)
        step(NBLK - 2, 1, 0, False, True, False)
        step(NBLK - 1, 0, 1, False, False, False)
        wait_rows(hrows[0], ssems[0])
        plsc.subcore_barrier()
        pltpu.sync_copy(
            acc.at[pl.ds(s * ROWS_PT, ROWS_PT)],
            agg_hbm.at[pl.ds(r * NP + s * ROWS_PT, ROWS_PT)],
        )


def _matmul_bias_body(x_ref, w_ref, b_ref, o_ref):
    o_ref[...] = (
        jnp.dot(x_ref[...], w_ref[...], preferred_element_type=jnp.float32)
        + b_ref[...]
    )


def _rw_body(x0, x1, x2, x3, w_ref, b_ref, o0, o1, o2, o3):
    for x_ref, o_ref in ((x0, o0), (x1, o1), (x2, o2), (x3, o3)):
        o_ref[...] = (
            jnp.dot(x_ref[...], w_ref[...], preferred_element_type=jnp.float32)
            + b_ref[...]
        )


def _rw_call(ews, wT, b):
    BA = 2000
    blk = pl.BlockSpec((BA, D), lambda i: (i, 0))
    return pl.pallas_call(
        _rw_body,
        grid=(E // BA,),
        in_specs=[blk, blk, blk, blk,
                  pl.BlockSpec((D, D), lambda i: (0, 0)),
                  pl.BlockSpec((1, D), lambda i: (0, 0))],
        out_specs=[blk, blk, blk, blk],
        out_shape=[jax.ShapeDtypeStruct((E, D), jnp.float32)] * 4,
    )(*ews, wT, b)


def _h0_call(feat, wT, b):
    return pl.pallas_call(
        _matmul_bias_body,
        out_shape=jax.ShapeDtypeStruct((N, D), jnp.float32),
    )(feat, wT, b)


def _prep_body(h0_ref, dt_ref, sc_ref, hs_ref):
    sc = lax.rsqrt(jnp.maximum(dt_ref[...], 1.0))
    sc_ref[...] = sc
    h = h0_ref[...]
    for r in range(R):
        hs_ref[r, 0:N, :] = h * sc[0:N, 2 * r:2 * r + 1]


def _prep_call(h0, degs_t):
    return pl.pallas_call(
        _prep_body,
        out_shape=[
            jax.ShapeDtypeStruct((NP, 8), jnp.float32),
            jax.ShapeDtypeStruct((R, NP, D), jnp.float32),
        ],
    )(h0, degs_t)


_BN = 1000


def _combine_emb(agg_ref, w_ref, b_ref, sc_ref):
    sc = sc_ref[...]
    acc = jnp.zeros((_BN, D), jnp.float32)
    for r in range(R):
        acc = acc + (
            jnp.dot(agg_ref[r], w_ref[r], preferred_element_type=jnp.float32)
            * sc[:, 2 * r + 1:2 * r + 2]
        )
    return acc * (1.0 / R) + jnp.sum(b_ref[...], axis=0, keepdims=True) * (1.0 / R)


def _combine_mid_body(agg_ref, w_ref, b_ref, sc_ref, hs2_ref):
    emb = _combine_emb(agg_ref, w_ref, b_ref, sc_ref)
    sc = sc_ref[...]
    for r in range(R):
        hs2_ref[r] = emb * sc[:, 2 * r:2 * r + 1]


_COMBINE_IN_SPECS = [
    pl.BlockSpec((R, _BN, D), lambda i: (0, i, 0)),
    pl.BlockSpec((R, D, D), lambda i: (0, 0, 0)),
    pl.BlockSpec((R, D), lambda i: (0, 0)),
    pl.BlockSpec((_BN, 8), lambda i: (i, 0)),
]


def _combine_mid_call(agg3, W, b, scales):
    return pl.pallas_call(
        _combine_mid_body,
        grid=(N // _BN,),
        in_specs=_COMBINE_IN_SPECS,
        out_specs=pl.BlockSpec((R, _BN, D), lambda i: (0, i, 0)),
        out_shape=jax.ShapeDtypeStruct((R, NP, D), jnp.float32),
    )(agg3, W, b, scales)


def _combine_final_body(agg_ref, w_ref, b_ref, sc_ref, out_ref):
    out_ref[...] = _combine_emb(agg_ref, w_ref, b_ref, sc_ref)


def _combine_final_call(agg3, W, b, scales):
    return pl.pallas_call(
        _combine_final_body,
        grid=(N // _BN,),
        in_specs=_COMBINE_IN_SPECS,
        out_specs=pl.BlockSpec((_BN, D), lambda i: (i, 0)),
        out_shape=jax.ShapeDtypeStruct((N, D), jnp.float32),
    )(agg3, W, b, scales)


def kernel(feat, edge_index_r0, edge_index_r1, edge_index_r2, edge_index_r3,
           edge_weight_r0, edge_weight_r1, edge_weight_r2, edge_weight_r3,
           node_fc_W, node_fc_b, rela_fc_W, rela_fc_b, W1, b1, W2, b2):
    eis = [edge_index_r0, edge_index_r1, edge_index_r2, edge_index_r3]
    ews = [edge_weight_r0, edge_weight_r1, edge_weight_r2, edge_weight_r3]

    srcs = [eis[r][0].astype(jnp.int32) for r in range(R)]
    dsts = [eis[r][1].astype(jnp.int32) for r in range(R)]
    src_adj = jnp.concatenate([srcs[r] + r * NP for r in range(R)])
    dst_flat = jnp.concatenate(dsts)
    deg_idx = jnp.concatenate(
        [jnp.concatenate([srcs[r] + ((r // 2) * 2) * NP,
                          dsts[r] + ((r // 2) * 2 + 1) * NP])
         for r in range(R)])
    rw0, rw1, rw2, rw3 = _rw_call(ews, rela_fc_W.T, rela_fc_b[None, :])
    h0 = _h0_call(feat, node_fc_W.T, node_fc_b[None, :])
    degs_flat = _deg_kernel(deg_idx)
    degs_t = degs_flat.reshape(R * 2, NP).T
    scales, hs1 = _prep_call(h0, degs_t)

    agg1 = _edge_kernel(hs1.reshape(R * NP, D), rw0, rw1, rw2, rw3,
                        src_adj, dst_flat)
    hs2 = _combine_mid_call(agg1.reshape(R, NP, D), W1, b1, scales)
    agg2 = _edge_kernel(hs2.reshape(R * NP, D), rw0, rw1, rw2, rw3,
                        src_adj, dst_flat)
    return _combine_final_call(agg2.reshape(R, NP, D), W2, b2, scales)
